# baseline (device time: 16089 ns/iter reference)
import jax
import jax.numpy as jnp
from jax import lax
from jax.experimental import pallas as pl
from jax.experimental.pallas import tpu as pltpu


def kernel(partial, resid, gamma):
    m, d = resid.shape
    half = m // 2
    gamma2 = gamma.reshape(1, d)

    def body(partial_ref, resid_ref, gamma_ref, out_ref,
             send_x, recv_x, send_y, recv_y,
             send_sem_x, recv_sem_x, send_sem_y, recv_sem_y):
        my_x = lax.axis_index("x")
        my_y = lax.axis_index("y")
        my_z = lax.axis_index("z")
        xpartner = (1 - my_x, my_y, my_z)
        ypartner = (my_x, my_y + 1 - 2 * (my_y % 2), my_z)

        h = (my_x + my_y) % 2
        my_start = h * half
        other_start = (1 - h) * half

        send_x[...] = partial_ref[0, pl.ds(other_start, half), :].astype(
            jnp.bfloat16)

        barrier_sem = pltpu.get_barrier_semaphore()
        for nbr in (xpartner, ypartner):
            pl.semaphore_signal(
                barrier_sem, inc=1,
                device_id=nbr, device_id_type=pl.DeviceIdType.MESH,
            )
        pl.semaphore_wait(barrier_sem, 2)

        rdma_x = pltpu.make_async_remote_copy(
            src_ref=send_x, dst_ref=recv_x,
            send_sem=send_sem_x, recv_sem=recv_sem_x,
            device_id=xpartner, device_id_type=pl.DeviceIdType.MESH,
        )
        rdma_x.start()
        rdma_x.wait()

        y = (partial_ref[0, pl.ds(my_start, half), :]
             + recv_x[...].astype(jnp.float32)
             + resid_ref[pl.ds(my_start, half), :])
        ms = jnp.mean(y * y, axis=-1, keepdims=True)
        outv = y * lax.rsqrt(ms + 1e-6) * gamma_ref[...]
        out_ref[pl.ds(my_start, half), :] = outv
        send_y[...] = outv.astype(jnp.bfloat16)

        rdma_y = pltpu.make_async_remote_copy(
            src_ref=send_y, dst_ref=recv_y,
            send_sem=send_sem_y, recv_sem=recv_sem_y,
            device_id=ypartner, device_id_type=pl.DeviceIdType.MESH,
        )
        rdma_y.start()
        rdma_y.wait()

        out_ref[pl.ds(other_start, half), :] = recv_y[...].astype(jnp.float32)

    return pl.pallas_call(
        body,
        out_shape=jax.ShapeDtypeStruct((m, d), jnp.float32),
        in_specs=[
            pl.BlockSpec(memory_space=pltpu.VMEM),
            pl.BlockSpec(memory_space=pltpu.VMEM),
            pl.BlockSpec(memory_space=pltpu.VMEM),
        ],
        out_specs=pl.BlockSpec(memory_space=pltpu.VMEM),
        scratch_shapes=[
            pltpu.VMEM((half, d), jnp.bfloat16),
            pltpu.VMEM((half, d), jnp.bfloat16),
            pltpu.VMEM((half, d), jnp.bfloat16),
            pltpu.VMEM((half, d), jnp.bfloat16),
            pltpu.SemaphoreType.DMA,
            pltpu.SemaphoreType.DMA,
            pltpu.SemaphoreType.DMA,
            pltpu.SemaphoreType.DMA,
        ],
        compiler_params=pltpu.CompilerParams(collective_id=0),
    )(partial, resid, gamma2)


# device time: 14009 ns/iter; 1.1485x vs baseline; 1.1485x over previous
import jax
import jax.numpy as jnp
from jax import lax
from jax.experimental import pallas as pl
from jax.experimental.pallas import tpu as pltpu

K = 4


def kernel(partial, resid, gamma):
    m, d = resid.shape
    half = m // 2
    rows = half // K
    gamma2 = gamma.reshape(1, d)

    def body(partial_ref, resid_ref, gamma_ref, out_ref,
             send_x, recv_x, send_y, recv_y,
             send_sems_x, recv_sems_x, send_sems_y, recv_sems_y):
        my_x = lax.axis_index("x")
        my_y = lax.axis_index("y")
        my_z = lax.axis_index("z")
        xpartner = (1 - my_x, my_y, my_z)
        ypartner = (my_x, my_y + 1 - 2 * (my_y % 2), my_z)

        h = (my_x + my_y) % 2
        my_start = h * half
        other_start = (1 - h) * half

        barrier_sem = pltpu.get_barrier_semaphore()
        for nbr in (xpartner, ypartner):
            pl.semaphore_signal(
                barrier_sem, inc=1,
                device_id=nbr, device_id_type=pl.DeviceIdType.MESH,
            )
        pl.semaphore_wait(barrier_sem, 2)

        rdmas_x = []
        for k in range(K):
            send_x[k] = partial_ref[
                0, pl.ds(other_start + k * rows, rows), :
            ].astype(jnp.bfloat16)
            rdma = pltpu.make_async_remote_copy(
                src_ref=send_x.at[k], dst_ref=recv_x.at[k],
                send_sem=send_sems_x.at[k], recv_sem=recv_sems_x.at[k],
                device_id=xpartner, device_id_type=pl.DeviceIdType.MESH,
            )
            rdma.start()
            rdmas_x.append(rdma)

        rdmas_y = []
        for k in range(K):
            sl = pl.ds(my_start + k * rows, rows)
            rdmas_x[k].wait_recv()
            y = (partial_ref[0, sl, :]
                 + recv_x[k].astype(jnp.float32)
                 + resid_ref[sl, :])
            ms = jnp.mean(y * y, axis=-1, keepdims=True)
            outv = y * lax.rsqrt(ms + 1e-6) * gamma_ref[...]
            out_ref[sl, :] = outv
            send_y[k] = outv.astype(jnp.bfloat16)
            rdma = pltpu.make_async_remote_copy(
                src_ref=send_y.at[k], dst_ref=recv_y.at[k],
                send_sem=send_sems_y.at[k], recv_sem=recv_sems_y.at[k],
                device_id=ypartner, device_id_type=pl.DeviceIdType.MESH,
            )
            rdma.start()
            rdmas_y.append(rdma)

        for k in range(K):
            rdmas_y[k].wait_recv()
            out_ref[pl.ds(other_start + k * rows, rows), :] = (
                recv_y[k].astype(jnp.float32))

        for k in range(K):
            rdmas_x[k].wait_send()
            rdmas_y[k].wait_send()

    return pl.pallas_call(
        body,
        out_shape=jax.ShapeDtypeStruct((m, d), jnp.float32),
        in_specs=[
            pl.BlockSpec(memory_space=pltpu.VMEM),
            pl.BlockSpec(memory_space=pltpu.VMEM),
            pl.BlockSpec(memory_space=pltpu.VMEM),
        ],
        out_specs=pl.BlockSpec(memory_space=pltpu.VMEM),
        scratch_shapes=[
            pltpu.VMEM((K, rows, d), jnp.bfloat16),
            pltpu.VMEM((K, rows, d), jnp.bfloat16),
            pltpu.VMEM((K, rows, d), jnp.bfloat16),
            pltpu.VMEM((K, rows, d), jnp.bfloat16),
            pltpu.SemaphoreType.DMA((K,)),
            pltpu.SemaphoreType.DMA((K,)),
            pltpu.SemaphoreType.DMA((K,)),
            pltpu.SemaphoreType.DMA((K,)),
        ],
        compiler_params=pltpu.CompilerParams(collective_id=0),
    )(partial, resid, gamma2)


# device time: 8190 ns/iter; 1.9645x vs baseline; 1.7105x over previous
import jax
import jax.numpy as jnp
from jax import lax
from jax.experimental import pallas as pl
from jax.experimental.pallas import tpu as pltpu

K = 4


def kernel(partial, resid, gamma):
    m, d = resid.shape
    rows = m // K
    gamma2 = gamma.reshape(1, d)

    def body(partial_ref, resid_ref, gamma_ref, out_ref,
             send_buf, recv_buf, send_sems, recv_sems):
        my_x = lax.axis_index("x")
        my_y = lax.axis_index("y")
        my_z = lax.axis_index("z")
        xpartner = (1 - my_x, my_y, my_z)

        barrier_sem = pltpu.get_barrier_semaphore()
        pl.semaphore_signal(
            barrier_sem, inc=1,
            device_id=xpartner, device_id_type=pl.DeviceIdType.MESH,
        )
        pl.semaphore_wait(barrier_sem, 1)

        rdmas = []
        for k in range(K):
            sl = slice(k * rows, (k + 1) * rows)
            send_buf[k] = partial_ref[0, sl, :].astype(jnp.bfloat16)
            rdma = pltpu.make_async_remote_copy(
                src_ref=send_buf.at[k], dst_ref=recv_buf.at[k],
                send_sem=send_sems.at[k], recv_sem=recv_sems.at[k],
                device_id=xpartner, device_id_type=pl.DeviceIdType.MESH,
            )
            rdma.start()
            rdmas.append(rdma)

        for k in range(K):
            sl = slice(k * rows, (k + 1) * rows)
            rdmas[k].wait_recv()
            y = (partial_ref[0, sl, :]
                 + recv_buf[k].astype(jnp.float32)
                 + resid_ref[sl, :])
            ms = jnp.mean(y * y, axis=-1, keepdims=True)
            out_ref[sl, :] = y * lax.rsqrt(ms + 1e-6) * gamma_ref[...]

        for k in range(K):
            rdmas[k].wait_send()

    return pl.pallas_call(
        body,
        out_shape=jax.ShapeDtypeStruct((m, d), jnp.float32),
        in_specs=[
            pl.BlockSpec(memory_space=pltpu.VMEM),
            pl.BlockSpec(memory_space=pltpu.VMEM),
            pl.BlockSpec(memory_space=pltpu.VMEM),
        ],
        out_specs=pl.BlockSpec(memory_space=pltpu.VMEM),
        scratch_shapes=[
            pltpu.VMEM((K, rows, d), jnp.bfloat16),
            pltpu.VMEM((K, rows, d), jnp.bfloat16),
            pltpu.SemaphoreType.DMA((K,)),
            pltpu.SemaphoreType.DMA((K,)),
        ],
        compiler_params=pltpu.CompilerParams(collective_id=0),
    )(partial, resid, gamma2)
